# baseline (speedup 1.0000x reference)
from functools import partial

import jax
import jax.numpy as jnp
from jax import lax
from jax.experimental import pallas as pl
from jax.experimental.pallas import tpu as pltpu

_KN = 256


def _fused_kernel(H, Bc, T, tsz,
                  x_ref, x0_ref, wp_ref, bp_ref, wleaf_ref, br_ref, wacc_ref,
                  out_ref, lp_s, hc_s):
    c = pl.program_id(0)
    nt = T // tsz
    R = Bc * tsz
    E = x_ref.shape[2]
    H5 = 5 * H
    f32 = jnp.float32
    bf16 = jnp.bfloat16

    _ = pltpu.matmul_pop(0, (R, _KN), f32, 0)
    _ = pltpu.matmul_pop(64, (R, _KN), f32, 0)
    _ = pltpu.matmul_pop(0, (R, _KN), f32, 1)
    _ = pltpu.matmul_pop(64, (R, _KN), f32, 1)

    is_c0 = c == 0
    row = lax.broadcasted_iota(jnp.int32, (R, 1), 0)
    m0r = jnp.logical_and(row < tsz, is_c0)
    bmask = jnp.logical_and(
        lax.broadcasted_iota(jnp.int32, (Bc, 1, 1), 0) == 0, is_c0)
    for tt in range(nt):
        xb = x_ref[:, tt * tsz:(tt + 1) * tsz, :]
        x0b = x0_ref[tt * tsz:(tt + 1) * tsz, :]
        xb = jnp.where(bmask, x0b[None], xb)
        xf = xb.reshape(R, E).astype(bf16)
        for kt in range(E // _KN):
            pltpu.matmul_push_rhs(wp_ref[kt * _KN:(kt + 1) * _KN, :], kt, 0)
            pltpu.matmul_acc_lhs(0, xf[:, kt * _KN:(kt + 1) * _KN], 0,
                                 load_staged_rhs=kt)
        proj = pltpu.matmul_pop(0, (R, _KN), f32, 0) + bp_ref[...]
        cc = proj[:, :H]
        h = jax.nn.sigmoid(proj[:, H:]) * jnp.tanh(cc)
        h16 = h.astype(bf16)
        zero16 = jnp.zeros_like(h16)
        h_ext = jnp.concatenate(
            [jnp.where(m0r, h16, zero16), jnp.where(m0r, zero16, h16)],
            axis=1)
        pltpu.matmul_push_rhs(wleaf_ref[:, 0:_KN], 0, 1)
        pltpu.matmul_acc_lhs(0, h_ext, 1, load_staged_rhs=0)
        pltpu.matmul_push_rhs(wleaf_ref[:, _KN:2 * _KN], 1, 1)
        pltpu.matmul_acc_lhs(64, h_ext, 1, load_staged_rhs=1)
        pltpu.matmul_push_rhs(wleaf_ref[:, 2 * _KN:3 * _KN], 0, 0)
        pltpu.matmul_acc_lhs(64, h_ext, 0, load_staged_rhs=0)
        lp0 = pltpu.matmul_pop(0, (R, _KN), f32, 1)
        lp1 = pltpu.matmul_pop(64, (R, _KN), f32, 1)
        lp2 = pltpu.matmul_pop(64, (R, _KN), f32, 0)
        lp = (jnp.concatenate([lp0, lp1, lp2[:, :H]], axis=1)
              + jnp.where(m0r, br_ref[0:1, :], br_ref[1:2, :]))
        lp_s[tt * tsz:(tt + 1) * tsz] = (
            jnp.swapaxes(lp.reshape(Bc, tsz, H5), 0, 1))
        hc_s[tt * tsz:(tt + 1) * tsz] = jnp.swapaxes(
            jnp.concatenate([h, cc], axis=1).reshape(Bc, tsz, 2 * H), 0, 1)

    rowb = lax.broadcasted_iota(jnp.int32, (Bc, 1), 0)
    m0 = jnp.logical_and(rowb == 0, is_c0)
    hc0 = hc_s[0]
    h0 = hc0[:, :H].astype(bf16)
    c0 = hc0[:, H:]
    zero = jnp.zeros_like(h0)
    acc0 = jnp.concatenate(
        [jnp.where(m0, h0, zero), jnp.where(m0, zero, h0)], axis=1)

    pltpu.matmul_push_rhs(wacc_ref[:, _KN:2 * _KN], 0, 1)
    pltpu.matmul_acc_lhs(0, jnp.zeros((Bc, _KN), bf16), 1,
                         load_staged_rhs=0)
    _ = pltpu.matmul_pop(0, (Bc, _KN), f32, 1)

    def step(k, carry):
        acc_ext, c_acc = carry
        lpk = lp_s[pl.ds(k, 1)][0]
        c_leaf = hc_s[pl.ds(k, 1)][0][:, H:]
        pltpu.matmul_push_rhs(wacc_ref[:, 0:_KN], 0, 0)
        pltpu.matmul_acc_lhs(0, acc_ext, 0, load_staged_rhs=0)
        pltpu.matmul_acc_lhs(0, acc_ext, 1, load_staged_rhs=None)
        pltpu.matmul_push_rhs(wacc_ref[:, 2 * _KN:3 * _KN], 1, 0)
        pltpu.matmul_acc_lhs(8, acc_ext, 0, load_staged_rhs=1)
        p0 = pltpu.matmul_pop(0, (Bc, _KN), f32, 0)
        p1 = pltpu.matmul_pop(0, (Bc, _KN), f32, 1)
        p2 = pltpu.matmul_pop(8, (Bc, _KN), f32, 0)
        proj = jnp.concatenate([p0, p1, p2[:, :H]], axis=1) + lpk
        i_g = jax.nn.sigmoid(proj[:, :H])
        f_acc = jax.nn.sigmoid(proj[:, H:2 * H])
        f_leaf = jax.nn.sigmoid(proj[:, 2 * H:3 * H])
        g_g = jnp.tanh(proj[:, 3 * H:4 * H])
        o_g = jax.nn.sigmoid(proj[:, 4 * H:])
        c_n = f_acc * c_acc + f_leaf * c_leaf + i_g * g_g
        h_n = (o_g * jnp.tanh(c_n)).astype(bf16)
        acc_n = jnp.concatenate(
            [jnp.where(m0, h_n, zero), jnp.where(m0, zero, h_n)], axis=1)
        return (acc_n, c_n)

    acc_ext, _ = lax.fori_loop(1, T, step, (acc0, c0), unroll=5)
    accf = acc_ext.astype(jnp.float32)
    out_ref[...] = accf[:, :H] + accf[:, H:]


def _swapf(w, H):
    return jnp.concatenate(
        [w[:, :H], w[:, 2 * H:3 * H], w[:, H:2 * H], w[:, 3 * H:]], axis=1)


def kernel(x, w_proj, b_proj, wl, wr, br):
    B, T, E = x.shape
    H = wl.shape[0]
    num_chunks = 2 if B % 2 == 0 and B >= 2 else 1
    Bc = B // num_chunks
    x0f = jnp.flip(x[0], 0)
    bf = jnp.bfloat16
    wp = w_proj.astype(bf)
    npad = 3 * _KN - 5 * H
    pad = jnp.zeros((2 * H, npad), jnp.float32)
    wleaf = jnp.concatenate([wr, _swapf(wl, H)], axis=0)
    wleaf = jnp.concatenate([wleaf, pad], axis=1).astype(bf)
    wacc = jnp.concatenate([wl, _swapf(wr, H)], axis=0)
    wacc = jnp.concatenate([wacc, pad], axis=1).astype(bf)
    brp = jnp.concatenate([br, _swapf(br, H)], axis=0)
    tsz = 8 if T % 8 == 0 else 1

    out = pl.pallas_call(
        partial(_fused_kernel, H, Bc, T, tsz),
        grid=(num_chunks,),
        in_specs=[
            pl.BlockSpec((Bc, T, E), lambda c: (c, 0, 0)),
            pl.BlockSpec((T, E), lambda c: (0, 0)),
            pl.BlockSpec((E, 2 * H), lambda c: (0, 0)),
            pl.BlockSpec((1, 2 * H), lambda c: (0, 0)),
            pl.BlockSpec((2 * H, 3 * _KN), lambda c: (0, 0)),
            pl.BlockSpec((2, 5 * H), lambda c: (0, 0)),
            pl.BlockSpec((2 * H, 3 * _KN), lambda c: (0, 0)),
        ],
        out_specs=pl.BlockSpec((None, Bc, H), lambda c: (c, 0, 0)),
        out_shape=jax.ShapeDtypeStruct((num_chunks, Bc, H), jnp.float32),
        scratch_shapes=[
            pltpu.VMEM((T, Bc, 5 * H), jnp.float32),
            pltpu.VMEM((T, Bc, 2 * H), jnp.float32),
        ],
        compiler_params=pltpu.CompilerParams(
            dimension_semantics=("parallel",),
            vmem_limit_bytes=100 * 2 ** 20),
    )(x, x0f, wp, b_proj, wleaf, brp, wacc)
    return out.reshape(B, H)


# speedup vs baseline: 1.0560x; 1.0560x over previous
from functools import partial

import jax
import jax.numpy as jnp
from jax import lax
from jax.experimental import pallas as pl
from jax.experimental.pallas import tpu as pltpu

_KN = 256


def _fused_kernel(H, Bc, T, tsz,
                  x_ref, x0_ref, wp_ref, bp_ref, wleaf_ref, br_ref, wacc_ref,
                  out_ref, lp_s, hc_s):
    c = pl.program_id(0)
    nt = T // tsz
    R = Bc * tsz
    E = x_ref.shape[2]
    H5 = 5 * H
    f32 = jnp.float32
    bf16 = jnp.bfloat16

    _ = pltpu.matmul_pop(0, (R, _KN), f32, 0)
    _ = pltpu.matmul_pop(64, (R, _KN), f32, 0)
    _ = pltpu.matmul_pop(0, (R, _KN), f32, 1)
    _ = pltpu.matmul_pop(64, (R, _KN), f32, 1)

    is_c0 = c == 0
    row = lax.broadcasted_iota(jnp.int32, (R, 1), 0)
    m0r = jnp.logical_and(row < tsz, is_c0)
    bmask = jnp.logical_and(
        lax.broadcasted_iota(jnp.int32, (Bc, 1, 1), 0) == 0, is_c0)
    for tt in range(nt):
        xb = x_ref[:, tt * tsz:(tt + 1) * tsz, :]
        x0b = x0_ref[tt * tsz:(tt + 1) * tsz, :]
        xb = jnp.where(bmask, x0b[None], xb)
        xf = xb.reshape(R, E).astype(bf16)
        for kt in range(E // _KN):
            pltpu.matmul_push_rhs(wp_ref[kt * _KN:(kt + 1) * _KN, :], kt, 0)
            pltpu.matmul_acc_lhs(0, xf[:, kt * _KN:(kt + 1) * _KN], 0,
                                 load_staged_rhs=kt)
        proj = pltpu.matmul_pop(0, (R, _KN), f32, 0) + bp_ref[...]
        cc = proj[:, :H]
        h = jax.nn.sigmoid(proj[:, H:]) * jnp.tanh(cc)
        h16 = h.astype(bf16)
        zero16 = jnp.zeros_like(h16)
        h_ext = jnp.concatenate(
            [jnp.where(m0r, h16, zero16), jnp.where(m0r, zero16, h16)],
            axis=1)
        pltpu.matmul_push_rhs(wleaf_ref[:, 0:_KN], 0, 1)
        pltpu.matmul_acc_lhs(0, h_ext, 1, load_staged_rhs=0)
        pltpu.matmul_push_rhs(wleaf_ref[:, _KN:2 * _KN], 1, 1)
        pltpu.matmul_acc_lhs(64, h_ext, 1, load_staged_rhs=1)
        pltpu.matmul_push_rhs(wleaf_ref[:, 2 * _KN:3 * _KN], 0, 0)
        pltpu.matmul_acc_lhs(64, h_ext, 0, load_staged_rhs=0)
        lp0 = pltpu.matmul_pop(0, (R, _KN), f32, 1)
        lp1 = pltpu.matmul_pop(64, (R, _KN), f32, 1)
        lp2 = pltpu.matmul_pop(64, (R, _KN), f32, 0)
        lp = (jnp.concatenate([lp0, lp1, lp2[:, :H]], axis=1)
              + jnp.where(m0r, br_ref[0:1, :], br_ref[1:2, :]))
        lp_s[tt * tsz:(tt + 1) * tsz] = (
            jnp.swapaxes(lp.reshape(Bc, tsz, H5), 0, 1))
        hc_s[tt * tsz:(tt + 1) * tsz] = jnp.swapaxes(
            jnp.concatenate([h, cc], axis=1).reshape(Bc, tsz, 2 * H), 0, 1)

    rowb = lax.broadcasted_iota(jnp.int32, (Bc, 1), 0)
    m0 = jnp.logical_and(rowb == 0, is_c0)
    hc0 = hc_s[0]
    h0 = hc0[:, :H].astype(bf16)
    c0 = hc0[:, H:]
    zero = jnp.zeros_like(h0)
    acc0 = jnp.concatenate(
        [jnp.where(m0, h0, zero), jnp.where(m0, zero, h0)], axis=1)

    pltpu.matmul_push_rhs(wacc_ref[:, _KN:2 * _KN], 0, 1)
    pltpu.matmul_acc_lhs(0, jnp.zeros((Bc, _KN), bf16), 1,
                         load_staged_rhs=0)
    _ = pltpu.matmul_pop(0, (Bc, _KN), f32, 1)
    pltpu.matmul_push_rhs(wacc_ref[:, 0:_KN], 0, 0)
    pltpu.matmul_push_rhs(wacc_ref[:, 2 * _KN:3 * _KN], 1, 0)

    def step(k, carry):
        acc_ext, c_acc = carry
        lpk = lp_s[pl.ds(k, 1)][0]
        c_leaf = hc_s[pl.ds(k, 1)][0][:, H:]
        pltpu.matmul_acc_lhs(0, acc_ext, 0, load_staged_rhs=0)
        pltpu.matmul_acc_lhs(0, acc_ext, 1, load_staged_rhs=None)
        pltpu.matmul_acc_lhs(8, acc_ext, 0, load_staged_rhs=1)
        pltpu.matmul_push_rhs(wacc_ref[:, 0:_KN], 0, 0)
        pltpu.matmul_push_rhs(wacc_ref[:, 2 * _KN:3 * _KN], 1, 0)
        p0 = pltpu.matmul_pop(0, (Bc, _KN), f32, 0)
        p1 = pltpu.matmul_pop(0, (Bc, _KN), f32, 1)
        p2 = pltpu.matmul_pop(8, (Bc, _KN), f32, 0)
        proj = jnp.concatenate([p0, p1, p2[:, :H]], axis=1) + lpk
        i_g = jax.nn.sigmoid(proj[:, :H])
        f_acc = jax.nn.sigmoid(proj[:, H:2 * H])
        f_leaf = jax.nn.sigmoid(proj[:, 2 * H:3 * H])
        g_g = jnp.tanh(proj[:, 3 * H:4 * H])
        o_g = jax.nn.sigmoid(proj[:, 4 * H:])
        c_n = f_acc * c_acc + f_leaf * c_leaf + i_g * g_g
        h_n = (o_g * jnp.tanh(c_n)).astype(bf16)
        acc_n = jnp.concatenate(
            [jnp.where(m0, h_n, zero), jnp.where(m0, zero, h_n)], axis=1)
        return (acc_n, c_n)

    acc_ext, _ = lax.fori_loop(1, T, step, (acc0, c0), unroll=5)
    accf = acc_ext.astype(jnp.float32)
    out_ref[...] = accf[:, :H] + accf[:, H:]


def _swapf(w, H):
    return jnp.concatenate(
        [w[:, :H], w[:, 2 * H:3 * H], w[:, H:2 * H], w[:, 3 * H:]], axis=1)


def kernel(x, w_proj, b_proj, wl, wr, br):
    B, T, E = x.shape
    H = wl.shape[0]
    num_chunks = 2 if B % 2 == 0 and B >= 2 else 1
    Bc = B // num_chunks
    x0f = jnp.flip(x[0], 0)
    bf = jnp.bfloat16
    wp = w_proj.astype(bf)
    npad = 3 * _KN - 5 * H
    pad = jnp.zeros((2 * H, npad), jnp.float32)
    wleaf = jnp.concatenate([wr, _swapf(wl, H)], axis=0)
    wleaf = jnp.concatenate([wleaf, pad], axis=1).astype(bf)
    wacc = jnp.concatenate([wl, _swapf(wr, H)], axis=0)
    wacc = jnp.concatenate([wacc, pad], axis=1).astype(bf)
    brp = jnp.concatenate([br, _swapf(br, H)], axis=0)
    tsz = 8 if T % 8 == 0 else 1

    out = pl.pallas_call(
        partial(_fused_kernel, H, Bc, T, tsz),
        grid=(num_chunks,),
        in_specs=[
            pl.BlockSpec((Bc, T, E), lambda c: (c, 0, 0)),
            pl.BlockSpec((T, E), lambda c: (0, 0)),
            pl.BlockSpec((E, 2 * H), lambda c: (0, 0)),
            pl.BlockSpec((1, 2 * H), lambda c: (0, 0)),
            pl.BlockSpec((2 * H, 3 * _KN), lambda c: (0, 0)),
            pl.BlockSpec((2, 5 * H), lambda c: (0, 0)),
            pl.BlockSpec((2 * H, 3 * _KN), lambda c: (0, 0)),
        ],
        out_specs=pl.BlockSpec((None, Bc, H), lambda c: (c, 0, 0)),
        out_shape=jax.ShapeDtypeStruct((num_chunks, Bc, H), jnp.float32),
        scratch_shapes=[
            pltpu.VMEM((T, Bc, 5 * H), jnp.float32),
            pltpu.VMEM((T, Bc, 2 * H), jnp.float32),
        ],
        compiler_params=pltpu.CompilerParams(
            dimension_semantics=("parallel",),
            vmem_limit_bytes=100 * 2 ** 20),
    )(x, x0f, wp, b_proj, wleaf, brp, wacc)
    return out.reshape(B, H)
